# Initial kernel scaffold; baseline (speedup 1.0000x reference)
import jax
import jax.numpy as jnp
from jax.experimental import pallas as pl


def kernel(p0, p1, p2, y):
    raise NotImplementedError("write your pallas kernel here")


# speedup vs baseline: 4.5509x; 4.5509x over previous
import jax
import jax.numpy as jnp
from jax.experimental import pallas as pl

_NUM_CLS = 80


def _stage_a_kernel(p0_ref, p1_ref, p2_ref,
                    yc0_ref, yc1_ref, yc2_ref,
                    yb0_ref, yb1_ref, yb2_ref,
                    n0_ref, n1_ref, n2_ref,
                    spos_ref, splen_ref, sneg_ref, sbox_ref):
    cls_pos_sum = jnp.float32(0.0)
    plen = jnp.float32(0.0)
    negcnt = jnp.float32(0.0)
    box_sum = jnp.float32(0.0)
    triples = ((p0_ref, yc0_ref, yb0_ref, n0_ref),
               (p1_ref, yc1_ref, yb1_ref, n1_ref),
               (p2_ref, yc2_ref, yb2_ref, n2_ref))
    nc = _NUM_CLS + 1
    for p_ref, yc_ref, yb_ref, nout_ref in triples:
        x = p_ref[0]
        cls = x[:nc]
        box = x[nc:nc + 4]
        mx = jnp.max(cls, axis=0)
        e = jnp.exp(cls - mx[None])
        lse = mx + jnp.log(jnp.sum(e, axis=0))
        ycls = yc_ref[0]
        tgt = jnp.clip(ycls, 0, _NUM_CLS)
        cidx = jax.lax.broadcasted_iota(jnp.int32, cls.shape, 0)
        xt = jnp.sum(jnp.where(cidx == tgt[None], cls, 0.0), axis=0)
        cls_loss = lse - xt
        posf = (ycls > 0).astype(jnp.float32)
        negf = (ycls == 0).astype(jnp.float32)
        nout_ref[0] = cls_loss * negf
        cls_pos_sum += jnp.sum(cls_loss * posf)
        plen += jnp.sum(posf)
        negcnt += jnp.sum(negf)
        d = box - yb_ref[0]
        ab = jnp.abs(d)
        sl1 = jnp.where(ab < 1.0, 0.5 * d * d, ab - 0.5)
        box_sum += jnp.sum(sl1 * posf[None])
    spos_ref[...] = cls_pos_sum.reshape(1, 1, 1)
    splen_ref[...] = plen.reshape(1, 1, 1)
    sneg_ref[...] = negcnt.reshape(1, 1, 1)
    sbox_ref[...] = box_sum.reshape(1, 1, 1)


def _stage_b_kernel(neg_ref, spos_ref, splen_ref, sneg_ref, sbox_ref,
                    loss_ref, cls_ref, box_ref):
    v = neg_ref[...]
    b, n = v.shape
    cls_pos_sum = jnp.sum(spos_ref[...])
    plen = jnp.sum(splen_ref[...])
    negcnt = jnp.sum(sneg_ref[...])
    box_sum = jnp.sum(sbox_ref[...])
    nlen = jnp.minimum(plen * 3.0, negcnt)
    kk = jnp.clip(nlen, 1.0, float(n))

    vb = jax.lax.bitcast_convert_type(v, jnp.int32)
    lo = jnp.full((b, 1), -1, dtype=jnp.int32)
    hi = jnp.full((b, 1), 0x7F800000, dtype=jnp.int32)

    def body(_, carry):
        lo_, hi_ = carry
        mid = lo_ + (hi_ - lo_) // 2
        cnt = jnp.sum((vb > mid).astype(jnp.float32), axis=1, keepdims=True)
        pred = cnt < kk
        return (jnp.where(pred, lo_, mid), jnp.where(pred, mid, hi_))

    lo, hi = jax.lax.fori_loop(0, 31, body, (lo, hi))
    tbits = hi
    gt = vb > tbits
    sum_gt = jnp.sum(jnp.where(gt, v, 0.0), axis=1, keepdims=True)
    cnt_gt = jnp.sum(gt.astype(jnp.float32), axis=1, keepdims=True)
    t = jax.lax.bitcast_convert_type(tbits, jnp.float32)
    negtop = jnp.sum(sum_gt + (kk - cnt_gt) * t)
    negtop = jnp.where(nlen >= 0.5, negtop, 0.0)

    cls_total = (cls_pos_sum + negtop) / (plen + nlen + 1e-8)
    box_total = box_sum / (plen + 1e-8)
    loss_ref[...] = (cls_total + box_total).reshape(1, 1)
    cls_ref[...] = cls_total.reshape(1, 1)
    box_ref[...] = box_total.reshape(1, 1)


def kernel(p0, p1, p2, y):
    maps = (p0, p1, p2)
    batch = p0.shape[0]
    f32 = jnp.float32

    ycls_list, ybox_list = [], []
    off = 0
    for p in maps:
        h, w = p.shape[2], p.shape[3]
        ysl = y[:, off:off + w * h, :]
        off += w * h
        ycls_list.append(
            ysl[..., 0].astype(jnp.int32).reshape(batch, w, h)
            .transpose(0, 2, 1))
        ybox_list.append(
            ysl[..., 1:5].reshape(batch, w, h, 4).transpose(0, 3, 2, 1))

    in_specs = []
    out_specs = []
    out_shapes = []
    for p in maps:
        c, h, w = p.shape[1], p.shape[2], p.shape[3]
        in_specs.append(pl.BlockSpec((1, c, h, w), lambda i: (i, 0, 0, 0)))
    for p in maps:
        h, w = p.shape[2], p.shape[3]
        in_specs.append(pl.BlockSpec((1, h, w), lambda i: (i, 0, 0)))
    for p in maps:
        h, w = p.shape[2], p.shape[3]
        in_specs.append(pl.BlockSpec((1, 4, h, w), lambda i: (i, 0, 0, 0)))
    for p in maps:
        h, w = p.shape[2], p.shape[3]
        out_specs.append(pl.BlockSpec((1, h, w), lambda i: (i, 0, 0)))
        out_shapes.append(jax.ShapeDtypeStruct((batch, h, w), f32))
    for _ in range(4):
        out_specs.append(pl.BlockSpec((1, 1, 1), lambda i: (i, 0, 0)))
        out_shapes.append(jax.ShapeDtypeStruct((batch, 1, 1), f32))

    outs = pl.pallas_call(
        _stage_a_kernel,
        grid=(batch,),
        in_specs=in_specs,
        out_specs=out_specs,
        out_shape=out_shapes,
    )(*maps, *ycls_list, *ybox_list)

    n0, n1, n2, spos, splen, sneg, sbox = outs
    negflat = jnp.concatenate(
        [n0.reshape(batch, -1), n1.reshape(batch, -1),
         n2.reshape(batch, -1)], axis=1)

    loss, cls_total, box_total = pl.pallas_call(
        _stage_b_kernel,
        out_shape=[jax.ShapeDtypeStruct((1, 1), f32)] * 3,
    )(negflat, spos, splen, sneg, sbox)

    return (loss[0, 0], cls_total[0, 0], box_total[0, 0])


# speedup vs baseline: 4.9930x; 1.0971x over previous
import jax
import jax.numpy as jnp
from jax.experimental import pallas as pl

_NUM_CLS = 80


def _stage_a_kernel(p0_ref, p1_ref, p2_ref,
                    yc0_ref, yc1_ref, yc2_ref,
                    yb0_ref, yb1_ref, yb2_ref,
                    n0_ref, n1_ref, n2_ref,
                    spos_ref, splen_ref, sneg_ref, sbox_ref):
    cls_pos_sum = jnp.float32(0.0)
    plen = jnp.float32(0.0)
    negcnt = jnp.float32(0.0)
    box_sum = jnp.float32(0.0)
    triples = ((p0_ref, yc0_ref, yb0_ref, n0_ref),
               (p1_ref, yc1_ref, yb1_ref, n1_ref),
               (p2_ref, yc2_ref, yb2_ref, n2_ref))
    nc = _NUM_CLS + 1
    for p_ref, yc_ref, yb_ref, nout_ref in triples:
        ycls = yc_ref[0]
        tgt = jnp.clip(ycls, 0, _NUM_CLS)
        mx = p_ref[0, 0]
        for c in range(1, nc):
            mx = jnp.maximum(mx, p_ref[0, c])
        s = jnp.zeros_like(mx)
        xt = jnp.zeros_like(mx)
        for c in range(nc):
            xc = p_ref[0, c]
            s = s + jnp.exp(xc - mx)
            xt = xt + jnp.where(tgt == c, xc, 0.0)
        lse = mx + jnp.log(s)
        box = p_ref[0, nc:nc + 4]
        cls_loss = lse - xt
        posf = (ycls > 0).astype(jnp.float32)
        negf = (ycls == 0).astype(jnp.float32)
        nout_ref[0] = cls_loss * negf
        cls_pos_sum += jnp.sum(cls_loss * posf)
        plen += jnp.sum(posf)
        negcnt += jnp.sum(negf)
        d = box - yb_ref[0]
        ab = jnp.abs(d)
        sl1 = jnp.where(ab < 1.0, 0.5 * d * d, ab - 0.5)
        box_sum += jnp.sum(sl1 * posf[None])
    spos_ref[...] = cls_pos_sum.reshape(1, 1, 1)
    splen_ref[...] = plen.reshape(1, 1, 1)
    sneg_ref[...] = negcnt.reshape(1, 1, 1)
    sbox_ref[...] = box_sum.reshape(1, 1, 1)


def _stage_b_kernel(n0_ref, n1_ref, n2_ref,
                    spos_ref, splen_ref, sneg_ref, sbox_ref,
                    loss_ref, cls_ref, box_ref):
    vs = (n0_ref[...], n1_ref[...], n2_ref[...])
    b = vs[0].shape[0]
    n = sum(v.shape[1] * v.shape[2] for v in vs)
    cls_pos_sum = jnp.sum(spos_ref[...])
    plen = jnp.sum(splen_ref[...])
    negcnt = jnp.sum(sneg_ref[...])
    box_sum = jnp.sum(sbox_ref[...])
    nlen = jnp.minimum(plen * 3.0, negcnt)
    kk = jnp.clip(nlen, 1.0, float(n))

    vbs = tuple(jax.lax.bitcast_convert_type(v, jnp.int32) for v in vs)
    lo = jnp.full((b, 1, 1), -1, dtype=jnp.int32)
    hi = jnp.full((b, 1, 1), 0x7F800000, dtype=jnp.int32)

    def row_count(mid):
        cnt = jnp.zeros((b, 1, 1), dtype=jnp.float32)
        for vb in vbs:
            cnt += jnp.sum((vb > mid).astype(jnp.float32), axis=(1, 2),
                           keepdims=True)
        return cnt

    def body(_, carry):
        lo_, hi_ = carry
        mid = lo_ + (hi_ - lo_) // 2
        pred = row_count(mid) < kk
        return (jnp.where(pred, lo_, mid), jnp.where(pred, mid, hi_))

    lo, hi = jax.lax.fori_loop(0, 31, body, (lo, hi))
    tbits = hi
    sum_gt = jnp.zeros((b, 1, 1), dtype=jnp.float32)
    cnt_gt = jnp.zeros((b, 1, 1), dtype=jnp.float32)
    for v, vb in zip(vs, vbs):
        gt = vb > tbits
        sum_gt += jnp.sum(jnp.where(gt, v, 0.0), axis=(1, 2), keepdims=True)
        cnt_gt += jnp.sum(gt.astype(jnp.float32), axis=(1, 2), keepdims=True)
    t = jax.lax.bitcast_convert_type(tbits, jnp.float32)
    negtop = jnp.sum(sum_gt + (kk - cnt_gt) * t)
    negtop = jnp.where(nlen >= 0.5, negtop, 0.0)

    cls_total = (cls_pos_sum + negtop) / (plen + nlen + 1e-8)
    box_total = box_sum / (plen + 1e-8)
    loss_ref[...] = (cls_total + box_total).reshape(1, 1)
    cls_ref[...] = cls_total.reshape(1, 1)
    box_ref[...] = box_total.reshape(1, 1)


def kernel(p0, p1, p2, y):
    maps = (p0, p1, p2)
    batch = p0.shape[0]
    f32 = jnp.float32

    ycls_list, ybox_list = [], []
    off = 0
    for p in maps:
        h, w = p.shape[2], p.shape[3]
        ysl = y[:, off:off + w * h, :]
        off += w * h
        ycls_list.append(
            ysl[..., 0].astype(jnp.int32).reshape(batch, w, h)
            .transpose(0, 2, 1))
        ybox_list.append(
            ysl[..., 1:5].reshape(batch, w, h, 4).transpose(0, 3, 2, 1))

    in_specs = []
    out_specs = []
    out_shapes = []
    for p in maps:
        c, h, w = p.shape[1], p.shape[2], p.shape[3]
        in_specs.append(pl.BlockSpec((1, c, h, w), lambda i: (i, 0, 0, 0)))
    for p in maps:
        h, w = p.shape[2], p.shape[3]
        in_specs.append(pl.BlockSpec((1, h, w), lambda i: (i, 0, 0)))
    for p in maps:
        h, w = p.shape[2], p.shape[3]
        in_specs.append(pl.BlockSpec((1, 4, h, w), lambda i: (i, 0, 0, 0)))
    for p in maps:
        h, w = p.shape[2], p.shape[3]
        out_specs.append(pl.BlockSpec((1, h, w), lambda i: (i, 0, 0)))
        out_shapes.append(jax.ShapeDtypeStruct((batch, h, w), f32))
    for _ in range(4):
        out_specs.append(pl.BlockSpec((1, 1, 1), lambda i: (i, 0, 0)))
        out_shapes.append(jax.ShapeDtypeStruct((batch, 1, 1), f32))

    outs = pl.pallas_call(
        _stage_a_kernel,
        grid=(batch,),
        in_specs=in_specs,
        out_specs=out_specs,
        out_shape=out_shapes,
    )(*maps, *ycls_list, *ybox_list)

    n0, n1, n2, spos, splen, sneg, sbox = outs

    loss, cls_total, box_total = pl.pallas_call(
        _stage_b_kernel,
        out_shape=[jax.ShapeDtypeStruct((1, 1), f32)] * 3,
    )(n0, n1, n2, spos, splen, sneg, sbox)

    return (loss[0, 0], cls_total[0, 0], box_total[0, 0])
